# baseline (speedup 1.0000x reference)
import functools

import jax
import jax.numpy as jnp
from jax import lax
from jax.experimental import pallas as pl
from jax.experimental.pallas import tpu as pltpu
from jax.experimental.pallas import tpu_sc as plsc

_NUM_FIELDS = 26
_VOCAB = 100000
_EMBED_DIM = 32
_BATCH = 4096

_FLAT_B = _BATCH * _NUM_FIELDS
_NUM_CORES = 2
_NUM_SUBCORES = 16
_NW = _NUM_CORES * _NUM_SUBCORES
_B_PER_W = _FLAT_B // _NW
_LANES = 16
_VECS = _B_PER_W // _LANES


def _make_gather():
    mesh = plsc.VectorSubcoreMesh(core_axis_name="c", subcore_axis_name="s")

    @functools.partial(
        pl.kernel,
        mesh=mesh,
        out_type=jax.ShapeDtypeStruct((_FLAT_B, _EMBED_DIM), jnp.float32),
        scratch_types=[
            pltpu.VMEM((_B_PER_W,), jnp.int32),
            pltpu.VMEM((_B_PER_W, _EMBED_DIM), jnp.float32),
            pltpu.SemaphoreType.DMA,
        ],
        compiler_params=pltpu.CompilerParams(use_tc_tiling_on_sc=False),
    )
    def gather_k(table_hbm, idx_hbm, out_hbm, idx_v, rows_v, sem):
        wid = lax.axis_index("s") * _NUM_CORES + lax.axis_index("c")
        base = wid * _B_PER_W
        pltpu.sync_copy(idx_hbm.at[pl.ds(base, _B_PER_W)], idx_v)

        def add_offset(j, carry):
            sl = pl.ds(j * _LANES, _LANES)
            pos = j * _LANES + lax.iota(jnp.int32, _LANES)
            fld = lax.rem(pos, _NUM_FIELDS)
            idx_v[sl] = idx_v[sl] + fld * _VOCAB
            return carry

        lax.fori_loop(0, _VECS, add_offset, 0)

        pltpu.async_copy(table_hbm.at[idx_v], rows_v, sem).wait()
        pltpu.sync_copy(rows_v, out_hbm.at[pl.ds(base, _B_PER_W)])

    return gather_k


_gather = _make_gather()


def kernel(x_cat, tables):
    flat_tables = tables.reshape(_NUM_FIELDS * _VOCAB, _EMBED_DIM)
    flat_idx = x_cat.astype(jnp.int32).reshape(_FLAT_B)
    out = _gather(flat_tables, flat_idx)
    return out.reshape(_BATCH, _NUM_FIELDS * _EMBED_DIM)


# speedup vs baseline: 5.7365x; 5.7365x over previous
import functools

import jax
import jax.numpy as jnp
from jax import lax
from jax.experimental import pallas as pl
from jax.experimental.pallas import tpu as pltpu
from jax.experimental.pallas import tpu_sc as plsc

_NUM_FIELDS = 26
_VOCAB = 100000
_EMBED_DIM = 32
_BATCH = 4096

_ROWS = _NUM_FIELDS * _EMBED_DIM
_NUM_CORES = 2
_NUM_SUBCORES = 16
_NW = _NUM_CORES * _NUM_SUBCORES
_LANES = 16
_BVECS = _BATCH // _LANES


def _make_gather():
    mesh = plsc.VectorSubcoreMesh(core_axis_name="c", subcore_axis_name="s")

    @functools.partial(
        pl.kernel,
        mesh=mesh,
        out_type=jax.ShapeDtypeStruct((_ROWS, _BATCH), jnp.float32),
        scratch_types=[
            pltpu.VMEM((_VOCAB,), jnp.float32),
            pltpu.VMEM((_BATCH,), jnp.int32),
            pltpu.VMEM((_BATCH,), jnp.float32),
        ],
        compiler_params=pltpu.CompilerParams(needs_layout_passes=False),
    )
    def gather_k(tab_hbm, idx_hbm, out_hbm, row_v, idx_v, out_v):
        w = lax.axis_index("s") * _NUM_CORES + lax.axis_index("c")

        def per_field(k, carry):
            r = k * _EMBED_DIM + w
            pltpu.sync_copy(idx_hbm.at[k], idx_v)
            pltpu.sync_copy(tab_hbm.at[r], row_v)

            def per_vec(j, c2):
                sl = pl.ds(j * _LANES, _LANES)
                out_v[sl] = plsc.load_gather(row_v, [idx_v[sl]])
                return c2

            lax.fori_loop(0, _BVECS, per_vec, 0)
            pltpu.sync_copy(out_v, out_hbm.at[r])
            return carry

        lax.fori_loop(0, _NUM_FIELDS, per_field, 0)

    return gather_k


_gather = _make_gather()


def kernel(x_cat, tables):
    tab2d = jnp.transpose(tables, (0, 2, 1)).reshape(_ROWS, _VOCAB)
    xt = jnp.transpose(x_cat.astype(jnp.int32))
    out_t = _gather(tab2d, xt)
    return jnp.transpose(out_t)


# speedup vs baseline: 5.7490x; 1.0022x over previous
import functools

import jax
import jax.numpy as jnp
from jax import lax
from jax.experimental import pallas as pl
from jax.experimental.pallas import tpu as pltpu
from jax.experimental.pallas import tpu_sc as plsc

_NUM_FIELDS = 26
_VOCAB = 100000
_EMBED_DIM = 32
_BATCH = 4096

_ROWS = _NUM_FIELDS * _EMBED_DIM
_NUM_CORES = 2
_NUM_SUBCORES = 16
_NW = _NUM_CORES * _NUM_SUBCORES
_LANES = 16
_BVECS = _BATCH // _LANES


def _make_gather():
    mesh = plsc.VectorSubcoreMesh(core_axis_name="c", subcore_axis_name="s")

    @functools.partial(
        pl.kernel,
        mesh=mesh,
        out_type=jax.ShapeDtypeStruct((_ROWS, _BATCH), jnp.float32),
        scratch_types=[
            pltpu.VMEM((_VOCAB,), jnp.float32),
            pltpu.VMEM((_BATCH,), jnp.int32),
            pltpu.VMEM((_BATCH,), jnp.float32),
        ],
        compiler_params=pltpu.CompilerParams(needs_layout_passes=False),
    )
    def gather_k(tab_hbm, idx_hbm, out_hbm, row_v, idx_v, out_v):
        w = lax.axis_index("c") * _NUM_SUBCORES + lax.axis_index("s")

        def per_field(k, carry):
            r = k * _EMBED_DIM + w
            pltpu.sync_copy(idx_hbm.at[k], idx_v)
            pltpu.sync_copy(tab_hbm.at[r], row_v)

            def per_vec(j, c2):
                sl = pl.ds(j * _LANES, _LANES)
                out_v[sl] = plsc.load_gather(row_v, [idx_v[sl]])
                return c2

            lax.fori_loop(0, _BVECS, per_vec, 0)
            pltpu.sync_copy(out_v, out_hbm.at[r])
            return carry

        lax.fori_loop(0, _NUM_FIELDS, per_field, 0)

    return gather_k


_gather = _make_gather()


def kernel(x_cat, tables):
    tab2d = jnp.transpose(tables, (0, 2, 1)).reshape(_ROWS, _VOCAB)
    xt = jnp.transpose(x_cat.astype(jnp.int32))
    out_t = _gather(tab2d, xt)
    return jnp.transpose(out_t)


# speedup vs baseline: 6.8302x; 1.1881x over previous
import functools

import jax
import jax.numpy as jnp
from jax import lax
from jax.experimental import pallas as pl
from jax.experimental.pallas import tpu as pltpu
from jax.experimental.pallas import tpu_sc as plsc

_NUM_FIELDS = 26
_VOCAB = 100000
_H0 = 50048
_H1 = _VOCAB - _H0
_EMBED_DIM = 32
_BATCH = 4096

_ROWS = _NUM_FIELDS * _EMBED_DIM
_NUM_CORES = 2
_NUM_SUBCORES = 16
_LANES = 16
_BVECS = _BATCH // _LANES


def _make_gather():
    mesh = plsc.VectorSubcoreMesh(core_axis_name="c", subcore_axis_name="s")

    @functools.partial(
        pl.kernel,
        mesh=mesh,
        out_type=jax.ShapeDtypeStruct((_ROWS, _BATCH), jnp.float32),
        scratch_types=[
            pltpu.VMEM((_H0,), jnp.float32),
            pltpu.VMEM((_H1,), jnp.float32),
            pltpu.VMEM((_BATCH,), jnp.int32),
            pltpu.VMEM((_BATCH,), jnp.float32),
            pltpu.SemaphoreType.DMA,
            pltpu.SemaphoreType.DMA,
        ],
        compiler_params=pltpu.CompilerParams(needs_layout_passes=False),
    )
    def gather_k(tab_hbm, idx_hbm, out_hbm, half_a, half_b, idx_v, out_v,
                 sem_a, sem_b):
        w = lax.axis_index("c") * _NUM_SUBCORES + lax.axis_index("s")

        def start_half(r, off, n, buf, sem):
            pltpu.make_async_copy(
                tab_hbm.at[r].at[pl.ds(off, n)], buf, sem).start()

        start_half(w, 0, _H0, half_a, sem_a)
        start_half(w, _H0, _H1, half_b, sem_b)
        pltpu.sync_copy(idx_hbm.at[0], idx_v)

        def per_field(k, carry):
            r = k * _EMBED_DIM + w
            r_next = r + _EMBED_DIM

            pltpu.make_async_copy(
                tab_hbm.at[r].at[pl.ds(0, _H0)], half_a, sem_a).wait()

            def pass_a(j, c2):
                sl = pl.ds(j * _LANES, _LANES)
                iv = idx_v[sl]
                m = iv < _H0
                out_v[sl] = plsc.load_gather(half_a, [iv], mask=m)
                return c2

            lax.fori_loop(0, _BVECS, pass_a, 0)

            @pl.when(k < _NUM_FIELDS - 1)
            def _():
                start_half(r_next, 0, _H0, half_a, sem_a)

            pltpu.make_async_copy(
                tab_hbm.at[r].at[pl.ds(_H0, _H1)], half_b, sem_b).wait()

            def pass_b(j, c2):
                sl = pl.ds(j * _LANES, _LANES)
                iv = idx_v[sl]
                m = iv >= _H0
                g = plsc.load_gather(half_b, [iv - _H0], mask=m)
                out_v[sl] = jnp.where(m, g, out_v[sl])
                return c2

            lax.fori_loop(0, _BVECS, pass_b, 0)

            @pl.when(k < _NUM_FIELDS - 1)
            def _():
                start_half(r_next, _H0, _H1, half_b, sem_b)

            pltpu.sync_copy(out_v, out_hbm.at[r])

            @pl.when(k < _NUM_FIELDS - 1)
            def _():
                pltpu.sync_copy(idx_hbm.at[k + 1], idx_v)

            return carry

        lax.fori_loop(0, _NUM_FIELDS, per_field, 0)

    return gather_k


_gather = _make_gather()


def kernel(x_cat, tables):
    tab2d = jnp.transpose(tables, (0, 2, 1)).reshape(_ROWS, _VOCAB)
    xt = jnp.transpose(x_cat.astype(jnp.int32))
    out_t = _gather(tab2d, xt)
    return jnp.transpose(out_t)


# speedup vs baseline: 7.0131x; 1.0268x over previous
import functools

import jax
import jax.numpy as jnp
from jax import lax
from jax.experimental import pallas as pl
from jax.experimental.pallas import tpu as pltpu
from jax.experimental.pallas import tpu_sc as plsc

_NUM_FIELDS = 26
_VOCAB = 100000
_H0 = 50048
_H1 = _VOCAB - _H0
_EMBED_DIM = 32
_BATCH = 4096

_ROWS = _NUM_FIELDS * _EMBED_DIM
_NUM_CORES = 2
_NUM_SUBCORES = 16
_LANES = 16
_UNROLL = 8
_BVECS = _BATCH // _LANES


def _make_gather():
    mesh = plsc.VectorSubcoreMesh(core_axis_name="c", subcore_axis_name="s")

    @functools.partial(
        pl.kernel,
        mesh=mesh,
        out_type=jax.ShapeDtypeStruct((_ROWS, _BATCH), jnp.float32),
        scratch_types=[
            pltpu.VMEM((_H0,), jnp.float32),
            pltpu.VMEM((_H1,), jnp.float32),
            pltpu.VMEM((_BATCH,), jnp.int32),
            pltpu.VMEM((_BATCH,), jnp.int32),
            pltpu.VMEM((_BATCH,), jnp.float32),
            pltpu.VMEM((_BATCH,), jnp.float32),
            pltpu.SemaphoreType.DMA,
            pltpu.SemaphoreType.DMA,
            pltpu.SemaphoreType.DMA,
            pltpu.SemaphoreType.DMA,
            pltpu.SemaphoreType.DMA,
            pltpu.SemaphoreType.DMA,
        ],
        compiler_params=pltpu.CompilerParams(needs_layout_passes=False),
    )
    def gather_k(tab_hbm, idx_hbm, out_hbm,
                 half_a, half_b, idx0, idx1, outv0, outv1,
                 sem_a, sem_b, sem_i0, sem_i1, sem_o0, sem_o1):
        w = lax.axis_index("c") * _NUM_SUBCORES + lax.axis_index("s")

        def row_half(r, off, n, buf, sem):
            return pltpu.make_async_copy(
                tab_hbm.at[r].at[pl.ds(off, n)], buf, sem)

        def idx_dma(k, buf, sem):
            return pltpu.make_async_copy(idx_hbm.at[k], buf, sem)

        def out_dma(r, buf, sem):
            return pltpu.make_async_copy(buf, out_hbm.at[r], sem)

        def pass_a(idx_v, out_v):
            def body(j, c2):
                for u in range(_UNROLL):
                    sl = pl.ds((j * _UNROLL + u) * _LANES, _LANES)
                    iv = idx_v[sl]
                    m = iv < _H0
                    out_v[sl] = plsc.load_gather(half_a, [iv], mask=m)
                return c2

            lax.fori_loop(0, _BVECS // _UNROLL, body, 0)

        def pass_b(idx_v, out_v):
            def body(j, c2):
                for u in range(_UNROLL):
                    sl = pl.ds((j * _UNROLL + u) * _LANES, _LANES)
                    iv = idx_v[sl]
                    m = iv >= _H0
                    g = plsc.load_gather(half_b, [iv - _H0], mask=m)
                    out_v[sl] = jnp.where(m, g, out_v[sl])
                return c2

            lax.fori_loop(0, _BVECS // _UNROLL, body, 0)

        row_half(w, 0, _H0, half_a, sem_a).start()
        row_half(w, _H0, _H1, half_b, sem_b).start()
        pltpu.sync_copy(idx_hbm.at[0], idx0)

        def field_pair(m, carry):
            k0 = m * 2
            k1 = k0 + 1
            r0 = k0 * _EMBED_DIM + w
            r1 = r0 + _EMBED_DIM

            @pl.when(m > 0)
            def _():
                out_dma(r0, outv0, sem_o0).wait()

            row_half(r0, 0, _H0, half_a, sem_a).wait()
            pass_a(idx0, outv0)

            @pl.when(k1 < _NUM_FIELDS)
            def _():
                row_half(r1, 0, _H0, half_a, sem_a).start()
                idx_dma(k1, idx1, sem_i1).start()

            row_half(r0, _H0, _H1, half_b, sem_b).wait()
            pass_b(idx0, outv0)

            @pl.when(k1 < _NUM_FIELDS)
            def _():
                row_half(r1, _H0, _H1, half_b, sem_b).start()

            out_dma(r0, outv0, sem_o0).start()

            @pl.when(m > 0)
            def _():
                out_dma(r1, outv1, sem_o1).wait()

            idx_dma(k1, idx1, sem_i1).wait()
            row_half(r1, 0, _H0, half_a, sem_a).wait()
            pass_a(idx1, outv1)

            @pl.when(k1 + 1 < _NUM_FIELDS)
            def _():
                row_half(r1 + _EMBED_DIM, 0, _H0, half_a, sem_a).start()
                idx_dma(k1 + 1, idx0, sem_i0).start()

            row_half(r1, _H0, _H1, half_b, sem_b).wait()
            pass_b(idx1, outv1)

            @pl.when(k1 + 1 < _NUM_FIELDS)
            def _():
                row_half(r1 + _EMBED_DIM, _H0, _H1, half_b, sem_b).start()

            out_dma(r1, outv1, sem_o1).start()

            @pl.when(k1 + 1 < _NUM_FIELDS)
            def _():
                idx_dma(k1 + 1, idx0, sem_i0).wait()

            return carry

        lax.fori_loop(0, _NUM_FIELDS // 2, field_pair, 0)
        out_dma(_ROWS - 2 * _EMBED_DIM + w, outv0, sem_o0).wait()
        out_dma(_ROWS - _EMBED_DIM + w, outv1, sem_o1).wait()

    return gather_k


_gather = _make_gather()


def kernel(x_cat, tables):
    tab2d = jnp.transpose(tables, (0, 2, 1)).reshape(_ROWS, _VOCAB)
    xt = jnp.transpose(x_cat.astype(jnp.int32))
    out_t = _gather(tab2d, xt)
    return jnp.transpose(out_t)


# speedup vs baseline: 7.9464x; 1.1331x over previous
import functools

import jax
import jax.numpy as jnp
from jax import lax
from jax.experimental import pallas as pl
from jax.experimental.pallas import tpu as pltpu
from jax.experimental.pallas import tpu_sc as plsc

_NUM_FIELDS = 26
_VOCAB = 100000
_T0 = 33408
_T1 = 33408
_T2 = _VOCAB - _T0 - _T1
_OFF1 = _T0
_OFF2 = _T0 + _T1
_EMBED_DIM = 32
_BATCH = 4096

_ROWS = _NUM_FIELDS * _EMBED_DIM
_NUM_CORES = 2
_NUM_SUBCORES = 16
_LANES = 16
_UNROLL = 8
_BVECS = _BATCH // _LANES


def _make_gather():
    mesh = plsc.VectorSubcoreMesh(core_axis_name="c", subcore_axis_name="s")

    @functools.partial(
        pl.kernel,
        mesh=mesh,
        out_type=jax.ShapeDtypeStruct((_ROWS, _BATCH), jnp.float32),
        scratch_types=[
            pltpu.VMEM((_T0,), jnp.float32),
            pltpu.VMEM((_T1,), jnp.float32),
            pltpu.VMEM((_T2,), jnp.float32),
            pltpu.VMEM((_BATCH,), jnp.int32),
            pltpu.VMEM((_BATCH,), jnp.int32),
            pltpu.VMEM((_BATCH,), jnp.float32),
            pltpu.VMEM((_BATCH,), jnp.float32),
            pltpu.SemaphoreType.DMA,
            pltpu.SemaphoreType.DMA,
            pltpu.SemaphoreType.DMA,
            pltpu.SemaphoreType.DMA,
            pltpu.SemaphoreType.DMA,
            pltpu.SemaphoreType.DMA,
            pltpu.SemaphoreType.DMA,
        ],
        compiler_params=pltpu.CompilerParams(needs_layout_passes=False),
    )
    def gather_k(tab_hbm, idx_hbm, out_hbm,
                 buf0, buf1, buf2, idx0, idx1, outv0, outv1,
                 sem0, sem1, sem2, sem_i0, sem_i1, sem_o0, sem_o1):
        w = lax.axis_index("c") * _NUM_SUBCORES + lax.axis_index("s")

        def third(r, off, n, buf, sem):
            return pltpu.make_async_copy(
                tab_hbm.at[r].at[pl.ds(off, n)], buf, sem)

        def idx_dma(k, buf, sem):
            return pltpu.make_async_copy(idx_hbm.at[k], buf, sem)

        def out_dma(r, buf, sem):
            return pltpu.make_async_copy(buf, out_hbm.at[r], sem)

        def pass0(idx_v, out_v):
            def body(j, c2):
                for u in range(_UNROLL):
                    sl = pl.ds((j * _UNROLL + u) * _LANES, _LANES)
                    iv = idx_v[sl]
                    m = iv < _T0
                    out_v[sl] = plsc.load_gather(buf0, [iv], mask=m)
                return c2

            lax.fori_loop(0, _BVECS // _UNROLL, body, 0)

        def pass1(idx_v, out_v):
            def body(j, c2):
                for u in range(_UNROLL):
                    sl = pl.ds((j * _UNROLL + u) * _LANES, _LANES)
                    d = idx_v[sl] - _OFF1
                    m = d.astype(jnp.uint32) < jnp.uint32(_T1)
                    g = plsc.load_gather(buf1, [d], mask=m)
                    out_v[sl] = jnp.where(m, g, out_v[sl])
                return c2

            lax.fori_loop(0, _BVECS // _UNROLL, body, 0)

        def pass2(idx_v, out_v):
            def body(j, c2):
                for u in range(_UNROLL):
                    sl = pl.ds((j * _UNROLL + u) * _LANES, _LANES)
                    d = idx_v[sl] - _OFF2
                    m = d >= 0
                    g = plsc.load_gather(buf2, [d], mask=m)
                    out_v[sl] = jnp.where(m, g, out_v[sl])
                return c2

            lax.fori_loop(0, _BVECS // _UNROLL, body, 0)

        def field(r, idx_v, out_v, has_next):
            pltpu.make_async_copy(
                tab_hbm.at[r].at[pl.ds(0, _T0)], buf0, sem0).wait()
            pass0(idx_v, out_v)

            @pl.when(has_next)
            def _():
                third(r + _EMBED_DIM, 0, _T0, buf0, sem0).start()

            pltpu.make_async_copy(
                tab_hbm.at[r].at[pl.ds(_OFF1, _T1)], buf1, sem1).wait()
            pass1(idx_v, out_v)

            @pl.when(has_next)
            def _():
                third(r + _EMBED_DIM, _OFF1, _T1, buf1, sem1).start()

            pltpu.make_async_copy(
                tab_hbm.at[r].at[pl.ds(_OFF2, _T2)], buf2, sem2).wait()
            pass2(idx_v, out_v)

            @pl.when(has_next)
            def _():
                third(r + _EMBED_DIM, _OFF2, _T2, buf2, sem2).start()

        third(w, 0, _T0, buf0, sem0).start()
        third(w, _OFF1, _T1, buf1, sem1).start()
        third(w, _OFF2, _T2, buf2, sem2).start()
        pltpu.sync_copy(idx_hbm.at[0], idx0)

        def field_pair(m, carry):
            k0 = m * 2
            k1 = k0 + 1
            r0 = k0 * _EMBED_DIM + w
            r1 = r0 + _EMBED_DIM

            @pl.when(m > 0)
            def _():
                out_dma(r0, outv0, sem_o0).wait()

            idx_dma(k1, idx1, sem_i1).start()
            field(r0, idx0, outv0, k1 < _NUM_FIELDS)
            out_dma(r0, outv0, sem_o0).start()

            @pl.when(m > 0)
            def _():
                out_dma(r1, outv1, sem_o1).wait()

            @pl.when(k1 + 1 < _NUM_FIELDS)
            def _():
                idx_dma(k1 + 1, idx0, sem_i0).start()

            idx_dma(k1, idx1, sem_i1).wait()
            field(r1, idx1, outv1, k1 + 1 < _NUM_FIELDS)
            out_dma(r1, outv1, sem_o1).start()

            @pl.when(k1 + 1 < _NUM_FIELDS)
            def _():
                idx_dma(k1 + 1, idx0, sem_i0).wait()

            return carry

        lax.fori_loop(0, _NUM_FIELDS // 2, field_pair, 0)
        out_dma(_ROWS - 2 * _EMBED_DIM + w, outv0, sem_o0).wait()
        out_dma(_ROWS - _EMBED_DIM + w, outv1, sem_o1).wait()

    return gather_k


_gather = _make_gather()


def kernel(x_cat, tables):
    tab2d = jnp.transpose(tables, (0, 2, 1)).reshape(_ROWS, _VOCAB)
    xt = jnp.transpose(x_cat.astype(jnp.int32))
    out_t = _gather(tab2d, xt)
    return jnp.transpose(out_t)
